# Initial kernel scaffold; baseline (speedup 1.0000x reference)
import jax
import jax.numpy as jnp
from jax.experimental import pallas as pl


def kernel(x, W_fused, W0, W1, P0, P1):
    raise NotImplementedError("write your pallas kernel here")


# speedup vs baseline: 5.6583x; 5.6583x over previous
import functools

import jax
import jax.numpy as jnp
from jax.experimental import pallas as pl
from jax.experimental.pallas import tpu as pltpu

_H = 256
_D = 1024
_T = 64
_MAX_ADAPTIVE_LR = 0.1


def _sig(z):
    return jax.nn.sigmoid(z)


def _nm_kernel(x_ref, wt_ref, w0_ref, p0_ref, w1t_ref, p1t_ref, out_ref,
               act_ref, s_ref, a0_ref, b1_ref, m0_ref, mb_ref):
    xb = x_ref[0]
    fp = jnp.dot(xb, wt_ref[...],
                 preferred_element_type=jnp.float32)
    q = fp[:, 0:_H]
    k = fp[:, _H:2 * _H]
    v = fp[:, 2 * _H:3 * _H]
    scal = fp[:, 3 * _H:3 * _H + 128]

    q = q * _sig(q)
    k = k * _sig(k)
    v = v * _sig(v)
    qn = jnp.sqrt(jnp.sum(q * q, axis=1, keepdims=True))
    kn = jnp.sqrt(jnp.sum(k * k, axis=1, keepdims=True))
    q = q / jnp.maximum(qn, 1e-12)
    k = k / jnp.maximum(kn, 1e-12)

    lane = jax.lax.broadcasted_iota(jnp.int32, (1, 128), 1)
    sgate = jnp.where(lane == 1, _MAX_ADAPTIVE_LR, 1.0)
    scal = _sig(scal) * sgate

    act_ref[:, 0:_H] = q
    act_ref[:, _H:2 * _H] = k
    act_ref[:, 2 * _H:3 * _H] = v
    s_ref[...] = scal

    a0_ref[...] = w0_ref[...] + p0_ref[...]
    b1_ref[...] = w1t_ref[...] + p1t_ref[...]
    m0_ref[...] = jnp.zeros((_H, _D), jnp.float32)
    mb_ref[...] = jnp.zeros((_H, _D), jnp.float32)

    inv_h2 = 2.0 / _H

    def step(t, carry):
        row = act_ref[pl.ds(t, 1), :]
        srow = s_ref[pl.ds(t, 1), :]
        q_r = row[:, 0:_H]
        k_r = row[:, _H:2 * _H]
        v_r = row[:, 2 * _H:3 * _H]
        a_t = srow[:, 0:1]
        th_t = srow[:, 1:2]
        e_t = srow[:, 2:3]

        a0 = a0_ref[...]
        b1 = b1_ref[...]

        u = jnp.dot(k_r, a0, preferred_element_type=jnp.float32)
        su = _sig(u)
        h = u * su
        pred = jax.lax.dot_general(h, b1, (((1,), (1,)), ((), ())),
                                   preferred_element_type=jnp.float32)
        dpred = inv_h2 * (pred - v_r)
        dh = jnp.dot(dpred, b1, preferred_element_type=jnp.float32)
        dpre = dh * (su * (1.0 + u * (1.0 - su)))

        g0 = jax.lax.dot_general(k_r, dpre, (((0,), (0,)), ((), ())),
                                 preferred_element_type=jnp.float32)
        g1 = jax.lax.dot_general(dpred, h, (((0,), (0,)), ((), ())),
                                 preferred_element_type=jnp.float32)

        two_th = 2.0 * th_t
        keep = 1.0 - a_t
        m0 = e_t * m0_ref[...] - two_th * g0
        a0n = keep * a0 + m0
        m0_ref[...] = m0
        a0_ref[...] = a0n
        mb = e_t * mb_ref[...] - two_th * g1
        b1n = keep * b1 + mb
        mb_ref[...] = mb
        b1_ref[...] = b1n

        u2 = jnp.dot(q_r, a0n, preferred_element_type=jnp.float32)
        h2 = u2 * _sig(u2)
        o = jax.lax.dot_general(h2, b1n, (((1,), (1,)), ((), ())),
                                preferred_element_type=jnp.float32)
        out_ref[0, pl.ds(t, 1), :, :] = o.reshape(1, 1, _H)
        return carry

    jax.lax.fori_loop(0, _T, step, 0)


@jax.jit
def kernel(x, W_fused, W0, W1, P0, P1):
    B, T, H = x.shape
    D = W0.shape[1]

    wqkv_t = W_fused[:3 * H].T
    wscal_t = jnp.zeros((H, 128), W_fused.dtype).at[:, :3].set(
        W_fused[3 * H:3 * H + 3].T)
    wt = jnp.concatenate([wqkv_t, wscal_t], axis=1)

    grid = (B,)
    out = pl.pallas_call(
        _nm_kernel,
        grid=grid,
        in_specs=[
            pl.BlockSpec((1, T, H), lambda b: (b, 0, 0)),
            pl.BlockSpec((H, 3 * H + 128), lambda b: (0, 0)),
            pl.BlockSpec((H, D), lambda b: (0, 0)),
            pl.BlockSpec((H, D), lambda b: (0, 0)),
            pl.BlockSpec((H, D), lambda b: (0, 0)),
            pl.BlockSpec((H, D), lambda b: (0, 0)),
        ],
        out_specs=pl.BlockSpec((1, T, 1, H), lambda b: (b, 0, 0, 0)),
        out_shape=jax.ShapeDtypeStruct((B, T, 1, H), jnp.float32),
        scratch_shapes=[
            pltpu.VMEM((T, 3 * H), jnp.float32),
            pltpu.VMEM((T, 128), jnp.float32),
            pltpu.VMEM((H, D), jnp.float32),
            pltpu.VMEM((H, D), jnp.float32),
            pltpu.VMEM((H, D), jnp.float32),
            pltpu.VMEM((H, D), jnp.float32),
        ],
        compiler_params=pltpu.CompilerParams(
            dimension_semantics=("parallel",),
        ),
    )(x, wt, W0, P0, W1.T, P1.T)
    return out.reshape(B, T, H)


# speedup vs baseline: 6.0909x; 1.0765x over previous
import jax
import jax.numpy as jnp
from jax.experimental import pallas as pl
from jax.experimental.pallas import tpu as pltpu

_H = 256
_D = 1024
_T = 64
_G = 2
_MAX_ADAPTIVE_LR = 0.1


def _sig(z):
    return jax.nn.sigmoid(z)


def _nm_kernel(x_ref, wt_ref, w0_ref, p0_ref, w1t_ref, p1t_ref, out_ref,
               act_ref, s_ref, a0_ref, b1_ref, m0_ref, mb_ref):
    lane = jax.lax.broadcasted_iota(jnp.int32, (1, 128), 1)
    sgate = jnp.where(lane == 1, _MAX_ADAPTIVE_LR, 1.0)
    for i in range(_G):
        xb = x_ref[i]
        fp = jnp.dot(xb, wt_ref[...],
                     preferred_element_type=jnp.float32)
        q = fp[:, 0:_H]
        k = fp[:, _H:2 * _H]
        v = fp[:, 2 * _H:3 * _H]
        scal = fp[:, 3 * _H:3 * _H + 128]

        q = q * _sig(q)
        k = k * _sig(k)
        v = v * _sig(v)
        qn = jnp.sqrt(jnp.sum(q * q, axis=1, keepdims=True))
        kn = jnp.sqrt(jnp.sum(k * k, axis=1, keepdims=True))
        q = q / jnp.maximum(qn, 1e-12)
        k = k / jnp.maximum(kn, 1e-12)

        act_ref[i, :, 0:_H] = q
        act_ref[i, :, _H:2 * _H] = k
        act_ref[i, :, 2 * _H:3 * _H] = v
        s_ref[i] = _sig(scal) * sgate

        a0_ref[i] = w0_ref[...] + p0_ref[...]
        b1_ref[i] = w1t_ref[...] + p1t_ref[...]
        m0_ref[i] = jnp.zeros((_H, _D), jnp.float32)
        mb_ref[i] = jnp.zeros((_H, _D), jnp.float32)

    inv_h2 = 2.0 / _H

    def one_step(i, t):
        row = act_ref[i, pl.ds(t, 1), :]
        srow = s_ref[i, pl.ds(t, 1), :]
        q_r = row[:, 0:_H]
        k_r = row[:, _H:2 * _H]
        v_r = row[:, 2 * _H:3 * _H]
        a_t = srow[:, 0:1]
        th_t = srow[:, 1:2]
        e_t = srow[:, 2:3]

        a0 = a0_ref[i]
        b1 = b1_ref[i]

        u = jnp.dot(k_r, a0, preferred_element_type=jnp.float32)
        su = _sig(u)
        h = u * su
        pred = jax.lax.dot_general(h, b1, (((1,), (1,)), ((), ())),
                                   preferred_element_type=jnp.float32)
        dpred = inv_h2 * (pred - v_r)
        dh = jnp.dot(dpred, b1, preferred_element_type=jnp.float32)
        dpre = dh * (su * (1.0 + u * (1.0 - su)))

        g0 = jax.lax.dot_general(k_r, dpre, (((0,), (0,)), ((), ())),
                                 preferred_element_type=jnp.float32)
        g1 = jax.lax.dot_general(dpred, h, (((0,), (0,)), ((), ())),
                                 preferred_element_type=jnp.float32)

        two_th = 2.0 * th_t
        keep = 1.0 - a_t
        m0 = e_t * m0_ref[i] - two_th * g0
        a0n = keep * a0 + m0
        m0_ref[i] = m0
        a0_ref[i] = a0n
        mb = e_t * mb_ref[i] - two_th * g1
        b1n = keep * b1 + mb
        mb_ref[i] = mb
        b1_ref[i] = b1n

        u2 = jnp.dot(q_r, a0n, preferred_element_type=jnp.float32)
        h2 = u2 * _sig(u2)
        o = jax.lax.dot_general(h2, b1n, (((1,), (1,)), ((), ())),
                                preferred_element_type=jnp.float32)
        out_ref[i, pl.ds(t, 1), :, :] = o.reshape(1, 1, _H)

    def step(t, carry):
        for i in range(_G):
            one_step(i, t)
        return carry

    jax.lax.fori_loop(0, _T, step, 0)


@jax.jit
def kernel(x, W_fused, W0, W1, P0, P1):
    B, T, H = x.shape
    D = W0.shape[1]

    wqkv_t = W_fused[:3 * H].T
    wscal_t = jnp.zeros((H, 128), W_fused.dtype).at[:, :3].set(
        W_fused[3 * H:3 * H + 3].T)
    wt = jnp.concatenate([wqkv_t, wscal_t], axis=1)

    grid = (B // _G,)
    out = pl.pallas_call(
        _nm_kernel,
        grid=grid,
        in_specs=[
            pl.BlockSpec((_G, T, H), lambda g: (g, 0, 0)),
            pl.BlockSpec((H, 3 * H + 128), lambda g: (0, 0)),
            pl.BlockSpec((H, D), lambda g: (0, 0)),
            pl.BlockSpec((H, D), lambda g: (0, 0)),
            pl.BlockSpec((H, D), lambda g: (0, 0)),
            pl.BlockSpec((H, D), lambda g: (0, 0)),
        ],
        out_specs=pl.BlockSpec((_G, T, 1, H), lambda g: (g, 0, 0, 0)),
        out_shape=jax.ShapeDtypeStruct((B, T, 1, H), jnp.float32),
        scratch_shapes=[
            pltpu.VMEM((_G, T, 3 * _H), jnp.float32),
            pltpu.VMEM((_G, T, 128), jnp.float32),
            pltpu.VMEM((_G, _H, _D), jnp.float32),
            pltpu.VMEM((_G, _H, _D), jnp.float32),
            pltpu.VMEM((_G, _H, _D), jnp.float32),
            pltpu.VMEM((_G, _H, _D), jnp.float32),
        ],
        compiler_params=pltpu.CompilerParams(
            dimension_semantics=("parallel",),
        ),
    )(x, wt, W0, P0, W1.T, P1.T)
    return out.reshape(B, T, H)


# speedup vs baseline: 7.7394x; 1.2706x over previous
import jax
import jax.numpy as jnp
from jax.experimental import pallas as pl
from jax.experimental.pallas import tpu as pltpu

_H = 256
_D = 1024
_T = 64
_C = 8
_G = 2
_MAX_ADAPTIVE_LR = 0.1

_NT = (((1,), (1,)), ((), ()))
_TN = (((0,), (0,)), ((), ()))


def _sig(z):
    return jax.nn.sigmoid(z)


def _dot(a, b, dims=None):
    if dims is None:
        return jnp.dot(a, b, preferred_element_type=jnp.float32)
    return jax.lax.dot_general(a, b, dims, preferred_element_type=jnp.float32)


def _nm_kernel(x_ref, wt_ref, w0_ref, p0_ref, w1t_ref, p1t_ref, out_ref,
               act_ref, s_ref, a0_ref, b1_ref, m0_ref, mb_ref):
    lane = jax.lax.broadcasted_iota(jnp.int32, (1, 128), 1)
    sgate = jnp.where(lane == 1, _MAX_ADAPTIVE_LR, 1.0)
    for i in range(_G):
        xb = x_ref[i]
        fp = _dot(xb, wt_ref[...])
        q = fp[:, 0:_H]
        k = fp[:, _H:2 * _H]
        v = fp[:, 2 * _H:3 * _H]
        scal = fp[:, 3 * _H:3 * _H + 128]

        q = q * _sig(q)
        k = k * _sig(k)
        v = v * _sig(v)
        qn = jnp.sqrt(jnp.sum(q * q, axis=1, keepdims=True))
        kn = jnp.sqrt(jnp.sum(k * k, axis=1, keepdims=True))
        q = q / jnp.maximum(qn, 1e-12)
        k = k / jnp.maximum(kn, 1e-12)

        act_ref[i, :, 0:_H] = q
        act_ref[i, :, _H:2 * _H] = k
        act_ref[i, :, 2 * _H:3 * _H] = v
        s_ref[i] = _sig(scal) * sgate

        a0_ref[i] = w0_ref[...] + p0_ref[...]
        b1_ref[i] = w1t_ref[...] + p1t_ref[...]
        m0_ref[i] = jnp.zeros((_H, _D), jnp.float32)
        mb_ref[i] = jnp.zeros((_H, _D), jnp.float32)

    inv_h2 = 2.0 / _H
    one = jnp.ones((1, 1), jnp.float32)
    zero = jnp.zeros((1, 1), jnp.float32)
    zrow = jnp.zeros((1, _C), jnp.float32)
    zcol = jnp.zeros((_C, 1), jnp.float32)
    eye_r = [(jax.lax.broadcasted_iota(jnp.int32, (1, _C), 1) == s
              ).astype(jnp.float32) for s in range(_C)]
    eye_c = [(jax.lax.broadcasted_iota(jnp.int32, (_C, 1), 0) == s
              ).astype(jnp.float32) for s in range(_C)]

    def chunk(i, t0):
        kc = act_ref[i, pl.ds(t0, _C), _H:2 * _H]
        qc = act_ref[i, pl.ds(t0, _C), 0:_H]
        vc = act_ref[i, pl.ds(t0, _C), 2 * _H:3 * _H]
        sc = s_ref[i, pl.ds(t0, _C), :]

        u0 = _dot(kc, a0_ref[i])
        um = _dot(kc, m0_ref[i])
        q0 = _dot(qc, a0_ref[i])
        qm = _dot(qc, m0_ref[i])
        kk = _dot(kc, kc, _NT)
        qk = _dot(qc, kc, _NT)

        lam, sigc, en = one, zero, one
        beta_r, m_r = zrow, zrow
        beta_c, m_c = zcol, zcol
        dpre_l, h_l, dp_l, h2_l = [], [], [], []
        lam_l, sig_l, brow_l = [], [], []

        for s in range(_C):
            v_r = vc[s:s + 1, :]
            a_t = sc[s:s + 1, 0:1]
            th_t = sc[s:s + 1, 1:2]
            e_t = sc[s:s + 1, 2:3]
            keep = 1.0 - a_t
            c_t = 2.0 * th_t

            u = lam * u0[s:s + 1, :] + sigc * um[s:s + 1, :]
            if s:
                u += _dot(beta_r[:, :s] * kk[s:s + 1, :s],
                          jnp.concatenate(dpre_l, axis=0))
            su = _sig(u)
            h = u * su

            pred = lam * _dot(h, b1_ref[i], _NT) + sigc * _dot(h, mb_ref[i], _NT)
            if s:
                hg = _dot(h, jnp.concatenate(h_l, axis=0), _NT)
                pred += _dot(hg * beta_r[:, :s], jnp.concatenate(dp_l, axis=0))
            dpred = inv_h2 * (pred - v_r)

            dh = lam * _dot(dpred, b1_ref[i]) + sigc * _dot(dpred, mb_ref[i])
            if s:
                dg = _dot(dpred, jnp.concatenate(dp_l, axis=0), _NT)
                dh += _dot(dg * beta_r[:, :s], jnp.concatenate(h_l, axis=0))
            dpre = dh * (su * (1.0 + u * (1.0 - su)))

            en = e_t * en
            m_r = e_t * m_r - c_t * eye_r[s]
            m_c = e_t * m_c - c_t * eye_c[s]
            beta_r = keep * beta_r + m_r
            beta_c = keep * beta_c + m_c
            sigc = keep * sigc + en
            lam = keep * lam

            dpre_l.append(dpre)
            h_l.append(h)
            dp_l.append(dpred)

            u2 = (lam * q0[s:s + 1, :] + sigc * qm[s:s + 1, :]
                  + _dot(beta_r[:, :s + 1] * qk[s:s + 1, :s + 1],
                         jnp.concatenate(dpre_l, axis=0)))
            h2 = u2 * _sig(u2)
            h2_l.append(h2)
            lam_l.append(lam)
            sig_l.append(sigc)
            brow_l.append(beta_r)

        dpre_m = jnp.concatenate(dpre_l, axis=0)
        h_m = jnp.concatenate(h_l, axis=0)
        dp_m = jnp.concatenate(dp_l, axis=0)
        h2_m = jnp.concatenate(h2_l, axis=0)
        lam_col = jnp.concatenate(lam_l, axis=0).reshape(_C, 1)
        sig_col = jnp.concatenate(sig_l, axis=0).reshape(_C, 1)
        bmat = jnp.concatenate(brow_l, axis=0)

        outs = (lam_col * _dot(h2_m, b1_ref[i], _NT)
                + sig_col * _dot(h2_m, mb_ref[i], _NT)
                + _dot(_dot(h2_m, h_m, _NT) * bmat, dp_m))
        out_ref[i, pl.ds(t0, _C), :, :] = outs.reshape(_C, 1, _H)

        a_new = (lam * a0_ref[i] + sigc * m0_ref[i]
                 + _dot(kc * beta_c, dpre_m, _TN))
        m_new = en * m0_ref[i] + _dot(kc * m_c, dpre_m, _TN)
        a0_ref[i] = a_new
        m0_ref[i] = m_new
        b_new = (lam * b1_ref[i] + sigc * mb_ref[i]
                 + _dot(dp_m * beta_c, h_m, _TN))
        mb_new = en * mb_ref[i] + _dot(dp_m * m_c, h_m, _TN)
        b1_ref[i] = b_new
        mb_ref[i] = mb_new

    def step(c, carry):
        t0 = c * _C
        for i in range(_G):
            chunk(i, t0)
        return carry

    jax.lax.fori_loop(0, _T // _C, step, 0)


@jax.jit
def kernel(x, W_fused, W0, W1, P0, P1):
    B, T, H = x.shape
    D = W0.shape[1]

    wqkv_t = W_fused[:3 * H].T
    wscal_t = jnp.zeros((H, 128), W_fused.dtype).at[:, :3].set(
        W_fused[3 * H:3 * H + 3].T)
    wt = jnp.concatenate([wqkv_t, wscal_t], axis=1)

    grid = (B // _G,)
    out = pl.pallas_call(
        _nm_kernel,
        grid=grid,
        in_specs=[
            pl.BlockSpec((_G, T, H), lambda g: (g, 0, 0)),
            pl.BlockSpec((H, 3 * H + 128), lambda g: (0, 0)),
            pl.BlockSpec((H, D), lambda g: (0, 0)),
            pl.BlockSpec((H, D), lambda g: (0, 0)),
            pl.BlockSpec((H, D), lambda g: (0, 0)),
            pl.BlockSpec((H, D), lambda g: (0, 0)),
        ],
        out_specs=pl.BlockSpec((_G, T, 1, H), lambda g: (g, 0, 0, 0)),
        out_shape=jax.ShapeDtypeStruct((B, T, 1, H), jnp.float32),
        scratch_shapes=[
            pltpu.VMEM((_G, T, 3 * _H), jnp.float32),
            pltpu.VMEM((_G, T, 128), jnp.float32),
            pltpu.VMEM((_G, _H, _D), jnp.float32),
            pltpu.VMEM((_G, _H, _D), jnp.float32),
            pltpu.VMEM((_G, _H, _D), jnp.float32),
            pltpu.VMEM((_G, _H, _D), jnp.float32),
        ],
        compiler_params=pltpu.CompilerParams(
            dimension_semantics=("parallel",),
        ),
    )(x, wt, W0, P0, W1.T, P1.T)
    return out.reshape(B, T, H)


# speedup vs baseline: 9.1439x; 1.1815x over previous
import jax
import jax.numpy as jnp
from jax.experimental import pallas as pl
from jax.experimental.pallas import tpu as pltpu

_H = 256
_D = 1024
_T = 64
_C = 8
_G = 2
_MAX_ADAPTIVE_LR = 0.1

_NT = (((1,), (1,)), ((), ()))
_TN = (((0,), (0,)), ((), ()))


def _sig(z):
    return jax.nn.sigmoid(z)


def _dot(a, b, dims=None):
    if dims is None:
        return jnp.dot(a, b, preferred_element_type=jnp.float32)
    return jax.lax.dot_general(a, b, dims, preferred_element_type=jnp.float32)


def _wsum(w, rows):
    acc = w[:, 0:1] * rows[0]
    for j in range(1, len(rows)):
        acc += w[:, j:j + 1] * rows[j]
    return acc


def _nm_kernel(x_ref, wt_ref, w0_ref, p0_ref, w1t_ref, p1t_ref, out_ref,
               act_ref, s_ref, am_ref, bmr_ref, bml_ref):
    lane = jax.lax.broadcasted_iota(jnp.int32, (1, 128), 1)
    sgate = jnp.where(lane == 1, _MAX_ADAPTIVE_LR, 1.0)
    for i in range(_G):
        xb = x_ref[i]
        fp = _dot(xb, wt_ref[...])
        q = fp[:, 0:_H]
        k = fp[:, _H:2 * _H]
        v = fp[:, 2 * _H:3 * _H]
        scal = fp[:, 3 * _H:3 * _H + 128]

        q = q * _sig(q)
        k = k * _sig(k)
        v = v * _sig(v)
        qn = jnp.sqrt(jnp.sum(q * q, axis=1, keepdims=True))
        kn = jnp.sqrt(jnp.sum(k * k, axis=1, keepdims=True))
        q = q / jnp.maximum(qn, 1e-12)
        k = k / jnp.maximum(kn, 1e-12)

        act_ref[i, :, 0:_H] = q
        act_ref[i, :, _H:2 * _H] = k
        act_ref[i, :, 2 * _H:3 * _H] = v
        s_ref[i] = _sig(scal) * sgate

        a_init = w0_ref[...] + p0_ref[...]
        b_init = w1t_ref[...] + p1t_ref[...]
        zmat = jnp.zeros((_H, _D), jnp.float32)
        am_ref[i, :, 0:_D] = a_init
        am_ref[i, :, _D:2 * _D] = zmat
        bmr_ref[i, 0:_H, :] = b_init
        bmr_ref[i, _H:2 * _H, :] = zmat
        bml_ref[i, :, 0:_D] = b_init
        bml_ref[i, :, _D:2 * _D] = zmat

    inv_h2 = 2.0 / _H
    one = jnp.ones((1, 1), jnp.float32)
    zero = jnp.zeros((1, 1), jnp.float32)
    zrow = jnp.zeros((1, _C), jnp.float32)
    zcol = jnp.zeros((_C, 1), jnp.float32)
    eye_r = [(jax.lax.broadcasted_iota(jnp.int32, (1, _C), 1) == s
              ).astype(jnp.float32) for s in range(_C)]
    eye_c = [(jax.lax.broadcasted_iota(jnp.int32, (_C, 1), 0) == s
              ).astype(jnp.float32) for s in range(_C)]

    def chunk(i, t0):
        kc = act_ref[i, pl.ds(t0, _C), _H:2 * _H]
        qc = act_ref[i, pl.ds(t0, _C), 0:_H]
        vc = act_ref[i, pl.ds(t0, _C), 2 * _H:3 * _H]
        sc = s_ref[i, pl.ds(t0, _C), :]
        kq = jnp.concatenate([kc, qc], axis=0)

        uu = _dot(kq, am_ref[i])
        u0, um = uu[0:_C, 0:_D], uu[0:_C, _D:2 * _D]
        q0, qm = uu[_C:2 * _C, 0:_D], uu[_C:2 * _C, _D:2 * _D]
        gg = _dot(kq, kc, _NT)
        kk, qk = gg[0:_C, :], gg[_C:2 * _C, :]

        lam, sigc, en = one, zero, one
        beta_r, m_r = zrow, zrow
        beta_c, m_c = zcol, zcol
        dpre_l, h_l, dp_l, h2_l = [], [], [], []
        lam_l, sig_l, brow_l = [], [], []

        for s in range(_C):
            v_r = vc[s:s + 1, :]
            a_t = sc[s:s + 1, 0:1]
            th_t = sc[s:s + 1, 1:2]
            e_t = sc[s:s + 1, 2:3]
            keep = 1.0 - a_t
            c_t = 2.0 * th_t

            u = lam * u0[s:s + 1, :] + sigc * um[s:s + 1, :]
            if s:
                u += _wsum(beta_r * kk[s:s + 1, :], dpre_l)
            su = _sig(u)
            h = u * su

            pb = _dot(h, bmr_ref[i], _NT)
            pred = lam * pb[:, 0:_H] + sigc * pb[:, _H:2 * _H]
            if s:
                hg = _dot(h, jnp.concatenate(h_l, axis=0), _NT)
                pred += _wsum(hg * beta_r[:, :s], dp_l)
            dpred = inv_h2 * (pred - v_r)

            db = _dot(dpred, bml_ref[i])
            dh = lam * db[:, 0:_D] + sigc * db[:, _D:2 * _D]
            if s:
                dg = _dot(dpred, jnp.concatenate(dp_l, axis=0), _NT)
                dh += _wsum(dg * beta_r[:, :s], h_l)
            dpre = dh * (su * (1.0 + u * (1.0 - su)))

            en = e_t * en
            m_r = e_t * m_r - c_t * eye_r[s]
            m_c = e_t * m_c - c_t * eye_c[s]
            beta_r = keep * beta_r + m_r
            beta_c = keep * beta_c + m_c
            sigc = keep * sigc + en
            lam = keep * lam

            dpre_l.append(dpre)
            h_l.append(h)
            dp_l.append(dpred)

            u2 = (lam * q0[s:s + 1, :] + sigc * qm[s:s + 1, :]
                  + _wsum(beta_r * qk[s:s + 1, :], dpre_l))
            h2 = u2 * _sig(u2)
            h2_l.append(h2)
            lam_l.append(lam)
            sig_l.append(sigc)
            brow_l.append(beta_r)

        dpre_m = jnp.concatenate(dpre_l, axis=0)
        h_m = jnp.concatenate(h_l, axis=0)
        dp_m = jnp.concatenate(dp_l, axis=0)
        h2_m = jnp.concatenate(h2_l, axis=0)
        lam_col = jnp.concatenate(lam_l, axis=0).reshape(_C, 1)
        sig_col = jnp.concatenate(sig_l, axis=0).reshape(_C, 1)
        bmat = jnp.concatenate(brow_l, axis=0)

        ob = _dot(h2_m, bmr_ref[i], _NT)
        outs = (lam_col * ob[:, 0:_H] + sig_col * ob[:, _H:2 * _H]
                + _dot(_dot(h2_m, h_m, _NT) * bmat, dp_m))
        out_ref[i, pl.ds(t0, _C), :, :] = outs.reshape(_C, 1, _H)

        d1 = _dot(jnp.concatenate([kc * beta_c, kc * m_c], axis=1),
                  dpre_m, _TN)
        a_old = am_ref[i, :, 0:_D]
        m_old = am_ref[i, :, _D:2 * _D]
        a_new = lam * a_old + sigc * m_old + d1[0:_H, :]
        m_new = en * m_old + d1[_H:2 * _H, :]
        am_ref[i, :, 0:_D] = a_new
        am_ref[i, :, _D:2 * _D] = m_new

        d2 = _dot(jnp.concatenate([dp_m * beta_c, dp_m * m_c], axis=1),
                  h_m, _TN)
        b_old = bmr_ref[i, 0:_H, :]
        mb_old = bmr_ref[i, _H:2 * _H, :]
        b_new = lam * b_old + sigc * mb_old + d2[0:_H, :]
        mb_new = en * mb_old + d2[_H:2 * _H, :]
        bmr_ref[i, 0:_H, :] = b_new
        bmr_ref[i, _H:2 * _H, :] = mb_new
        bml_ref[i, :, 0:_D] = b_new
        bml_ref[i, :, _D:2 * _D] = mb_new

    def step(c, carry):
        t0 = c * _C
        for i in range(_G):
            chunk(i, t0)
        return carry

    jax.lax.fori_loop(0, _T // _C, step, 0)


@jax.jit
def kernel(x, W_fused, W0, W1, P0, P1):
    B, T, H = x.shape
    D = W0.shape[1]

    wqkv_t = W_fused[:3 * H].T
    wscal_t = jnp.zeros((H, 128), W_fused.dtype).at[:, :3].set(
        W_fused[3 * H:3 * H + 3].T)
    wt = jnp.concatenate([wqkv_t, wscal_t], axis=1)

    grid = (B // _G,)
    out = pl.pallas_call(
        _nm_kernel,
        grid=grid,
        in_specs=[
            pl.BlockSpec((_G, T, H), lambda g: (g, 0, 0)),
            pl.BlockSpec((H, 3 * H + 128), lambda g: (0, 0)),
            pl.BlockSpec((H, D), lambda g: (0, 0)),
            pl.BlockSpec((H, D), lambda g: (0, 0)),
            pl.BlockSpec((H, D), lambda g: (0, 0)),
            pl.BlockSpec((H, D), lambda g: (0, 0)),
        ],
        out_specs=pl.BlockSpec((_G, T, 1, H), lambda g: (g, 0, 0, 0)),
        out_shape=jax.ShapeDtypeStruct((B, T, 1, H), jnp.float32),
        scratch_shapes=[
            pltpu.VMEM((_G, T, 3 * _H), jnp.float32),
            pltpu.VMEM((_G, T, 128), jnp.float32),
            pltpu.VMEM((_G, _H, 2 * _D), jnp.float32),
            pltpu.VMEM((_G, 2 * _H, _D), jnp.float32),
            pltpu.VMEM((_G, _H, 2 * _D), jnp.float32),
        ],
        compiler_params=pltpu.CompilerParams(
            dimension_semantics=("parallel",),
        ),
    )(x, wt, W0, P0, W1.T, P1.T)
    return out.reshape(B, T, H)
